# baseline (speedup 1.0000x reference)
import functools

import jax
import jax.numpy as jnp
from jax import lax
from jax.experimental import pallas as pl
from jax.experimental.pallas import tpu as pltpu
from jax.experimental.pallas import tpu_sc as plsc

NUM_ITEMS = 100000
PAD_IDX = NUM_ITEMS
D = 128
B = 1024
L = 50

NSEG = 1
LSEG = L // NSEG
ROWS_SEG = LSEG * B

NUM_WORKERS = 32
ROWS_PER_W = ROWS_SEG // NUM_WORKERS
CHUNK = 80
NCHUNK = ROWS_PER_W // CHUNK


def _sc_gather_body(seq_hbm, table_hbm, out_hbm, idx_all, rows0, rows1,
                    sem0, sem1):
    c = lax.axis_index("c")
    s = lax.axis_index("s")
    wid = s * 2 + c
    base = wid * ROWS_PER_W
    pltpu.sync_copy(seq_hbm.at[pl.ds(base, ROWS_PER_W)], idx_all)
    bufs = (rows0, rows1)
    sems = (sem0, sem1)

    def start(ch):
        return pltpu.async_copy(
            table_hbm.at[idx_all.at[pl.ds(ch * CHUNK, CHUNK)]],
            bufs[ch % 2], sems[ch % 2])

    cps = [None] * NCHUNK
    cps[0] = start(0)
    for ch in range(NCHUNK):
        if ch + 1 < NCHUNK:
            cps[ch + 1] = start(ch + 1)
        cps[ch].wait()
        pltpu.sync_copy(bufs[ch % 2],
                        out_hbm.at[pl.ds(base + ch * CHUNK, CHUNK)])


def _sc_gather(seq_flat_seg, table):
    mesh = plsc.VectorSubcoreMesh(core_axis_name="c", subcore_axis_name="s")
    return pl.kernel(
        _sc_gather_body,
        mesh=mesh,
        out_type=jax.ShapeDtypeStruct((ROWS_SEG, D), jnp.float32),
        scratch_types=[
            pltpu.VMEM((ROWS_PER_W,), jnp.int32),
            pltpu.VMEM((CHUNK, D), jnp.float32),
            pltpu.VMEM((CHUNK, D), jnp.float32),
            pltpu.SemaphoreType.DMA,
            pltpu.SemaphoreType.DMA,
        ],
    )(seq_flat_seg, table)


LC = 5


def _gru_body(emb_ref, h0_ref, wih_ref, whh_ref, bih_ref, bhh_ref, out_ref,
              h_ref):
    l = pl.program_id(0)

    @pl.when(l == 0)
    def _():
        h_ref[...] = h0_ref[...]

    h = h_ref[...]
    for t in range(LC):
        x_t = emb_ref[t].astype(jnp.bfloat16)
        gi = (
            jnp.dot(x_t, wih_ref[...], preferred_element_type=jnp.float32)
            + bih_ref[...]
        )
        gh = (
            jnp.dot(h.astype(jnp.bfloat16), whh_ref[...],
                    preferred_element_type=jnp.float32)
            + bhh_ref[...]
        )
        r = 0.5 + 0.5 * jnp.tanh(0.5 * (gi[:, :D] + gh[:, :D]))
        z = 0.5 + 0.5 * jnp.tanh(0.5 * (gi[:, D:2 * D] + gh[:, D:2 * D]))
        n = jnp.tanh(gi[:, 2 * D:] + r * gh[:, 2 * D:])
        h = n + z * (h - n)

    h_ref[...] = h
    out_ref[...] = h


def _gru(emb_lbd, h0, wih_t, whh_t, b_ih2, b_hh2):
    return pl.pallas_call(
        _gru_body,
        grid=(LSEG // LC,),
        in_specs=[
            pl.BlockSpec((LC, B, D), lambda l: (l, 0, 0)),
            pl.BlockSpec((B, D), lambda l: (0, 0)),
            pl.BlockSpec((D, 3 * D), lambda l: (0, 0)),
            pl.BlockSpec((D, 3 * D), lambda l: (0, 0)),
            pl.BlockSpec((1, 3 * D), lambda l: (0, 0)),
            pl.BlockSpec((1, 3 * D), lambda l: (0, 0)),
        ],
        out_specs=pl.BlockSpec((B, D), lambda l: (0, 0)),
        out_shape=jax.ShapeDtypeStruct((B, D), jnp.float32),
        scratch_shapes=[
            pltpu.VMEM((B, D), jnp.float32),
        ],
    )(emb_lbd, h0, wih_t, whh_t, b_ih2, b_hh2)


@jax.jit
def kernel(item_seq, item_table, W_ih, W_hh, b_ih, b_hh):
    seq = jnp.where(item_seq == -1, PAD_IDX, item_seq).astype(jnp.int32)
    seq_t = seq.T.reshape(L * B)
    wih_t = W_ih.T.astype(jnp.bfloat16)
    whh_t = W_hh.T.astype(jnp.bfloat16)
    b_ih2 = b_ih.reshape(1, 3 * D)
    b_hh2 = b_hh.reshape(1, 3 * D)
    embs = [
        _sc_gather(lax.slice(seq_t, (s * ROWS_SEG,), ((s + 1) * ROWS_SEG,)),
                   item_table)
        for s in range(NSEG)
    ]
    h = jnp.zeros((B, D), jnp.float32)
    for s in range(NSEG):
        h = _gru(embs[s].reshape(LSEG, B, D), h, wih_t, whh_t, b_ih2, b_hh2)
    return h


# speedup vs baseline: 1.1079x; 1.1079x over previous
import functools

import jax
import jax.numpy as jnp
from jax import lax
from jax.experimental import pallas as pl
from jax.experimental.pallas import tpu as pltpu
from jax.experimental.pallas import tpu_sc as plsc

NUM_ITEMS = 100000
PAD_IDX = NUM_ITEMS
D = 128
B = 1024
L = 50

NSEG = 2
LSEG = L // NSEG
ROWS_SEG = LSEG * B

NUM_WORKERS = 32
ROWS_PER_W = ROWS_SEG // NUM_WORKERS
CHUNK = 80
NCHUNK = ROWS_PER_W // CHUNK


def _sc_gather_body(seq_hbm, table_hbm, out_hbm, idx_all, rows0, rows1,
                    sem0, sem1):
    c = lax.axis_index("c")
    s = lax.axis_index("s")
    wid = s * 2 + c
    base = wid * ROWS_PER_W
    pltpu.sync_copy(seq_hbm.at[pl.ds(base, ROWS_PER_W)], idx_all)
    bufs = (rows0, rows1)
    sems = (sem0, sem1)

    def start(ch):
        return pltpu.async_copy(
            table_hbm.at[idx_all.at[pl.ds(ch * CHUNK, CHUNK)]],
            bufs[ch % 2], sems[ch % 2])

    cps = [None] * NCHUNK
    cps[0] = start(0)
    for ch in range(NCHUNK):
        if ch + 1 < NCHUNK:
            cps[ch + 1] = start(ch + 1)
        cps[ch].wait()
        pltpu.sync_copy(bufs[ch % 2],
                        out_hbm.at[pl.ds(base + ch * CHUNK, CHUNK)])


def _sc_gather(seq_flat_seg, table):
    mesh = plsc.VectorSubcoreMesh(core_axis_name="c", subcore_axis_name="s")
    return pl.kernel(
        _sc_gather_body,
        mesh=mesh,
        out_type=jax.ShapeDtypeStruct((ROWS_SEG, D), jnp.float32),
        scratch_types=[
            pltpu.VMEM((ROWS_PER_W,), jnp.int32),
            pltpu.VMEM((CHUNK, D), jnp.float32),
            pltpu.VMEM((CHUNK, D), jnp.float32),
            pltpu.SemaphoreType.DMA,
            pltpu.SemaphoreType.DMA,
        ],
    )(seq_flat_seg, table)


LC = 5


def _gru_body(emb_ref, h0_ref, wih_ref, whh_ref, bih_ref, bhh_ref, out_ref,
              h_ref):
    l = pl.program_id(0)

    @pl.when(l == 0)
    def _():
        h_ref[...] = h0_ref[...]

    h = h_ref[...]
    for t in range(LC):
        x_t = emb_ref[t].astype(jnp.bfloat16)
        gi = (
            jnp.dot(x_t, wih_ref[...], preferred_element_type=jnp.float32)
            + bih_ref[...]
        )
        gh = (
            jnp.dot(h.astype(jnp.bfloat16), whh_ref[...],
                    preferred_element_type=jnp.float32)
            + bhh_ref[...]
        )
        r = 0.5 + 0.5 * jnp.tanh(0.5 * (gi[:, :D] + gh[:, :D]))
        z = 0.5 + 0.5 * jnp.tanh(0.5 * (gi[:, D:2 * D] + gh[:, D:2 * D]))
        n = jnp.tanh(gi[:, 2 * D:] + r * gh[:, 2 * D:])
        h = n + z * (h - n)

    h_ref[...] = h
    out_ref[...] = h


def _gru(emb_lbd, h0, wih_t, whh_t, b_ih2, b_hh2):
    return pl.pallas_call(
        _gru_body,
        grid=(LSEG // LC,),
        in_specs=[
            pl.BlockSpec((LC, B, D), lambda l: (l, 0, 0)),
            pl.BlockSpec((B, D), lambda l: (0, 0)),
            pl.BlockSpec((D, 3 * D), lambda l: (0, 0)),
            pl.BlockSpec((D, 3 * D), lambda l: (0, 0)),
            pl.BlockSpec((1, 3 * D), lambda l: (0, 0)),
            pl.BlockSpec((1, 3 * D), lambda l: (0, 0)),
        ],
        out_specs=pl.BlockSpec((B, D), lambda l: (0, 0)),
        out_shape=jax.ShapeDtypeStruct((B, D), jnp.float32),
        scratch_shapes=[
            pltpu.VMEM((B, D), jnp.float32),
        ],
    )(emb_lbd, h0, wih_t, whh_t, b_ih2, b_hh2)


@jax.jit
def kernel(item_seq, item_table, W_ih, W_hh, b_ih, b_hh):
    seq = jnp.where(item_seq == -1, PAD_IDX, item_seq).astype(jnp.int32)
    seq_t = seq.T.reshape(L * B)
    wih_t = W_ih.T.astype(jnp.bfloat16)
    whh_t = W_hh.T.astype(jnp.bfloat16)
    b_ih2 = b_ih.reshape(1, 3 * D)
    b_hh2 = b_hh.reshape(1, 3 * D)
    embs = [
        _sc_gather(lax.slice(seq_t, (s * ROWS_SEG,), ((s + 1) * ROWS_SEG,)),
                   item_table)
        for s in range(NSEG)
    ]
    h = jnp.zeros((B, D), jnp.float32)
    for s in range(NSEG):
        h = _gru(embs[s].reshape(LSEG, B, D), h, wih_t, whh_t, b_ih2, b_hh2)
    return h


# speedup vs baseline: 1.1212x; 1.0120x over previous
import functools

import jax
import jax.numpy as jnp
from jax import lax
from jax.experimental import pallas as pl
from jax.experimental.pallas import tpu as pltpu
from jax.experimental.pallas import tpu_sc as plsc

NUM_ITEMS = 100000
PAD_IDX = NUM_ITEMS
D = 128
B = 1024
L = 50

NSEG = 2
LSEG = L // NSEG
ROWS_SEG = LSEG * B

NUM_WORKERS = 32
ROWS_PER_W = ROWS_SEG // NUM_WORKERS
CHUNK = 80
NCHUNK = ROWS_PER_W // CHUNK


NBUF = 4


def _sc_gather_body(seq_hbm, table_hbm, out_hbm, idx_all, rows0, rows1,
                    rows2, rows3, gsem0, gsem1, gsem2, gsem3,
                    ssem0, ssem1, ssem2, ssem3):
    c = lax.axis_index("c")
    s = lax.axis_index("s")
    wid = s * 2 + c
    base = wid * ROWS_PER_W
    pltpu.sync_copy(seq_hbm.at[pl.ds(base, ROWS_PER_W)], idx_all)
    bufs = (rows0, rows1, rows2, rows3)
    gsems = (gsem0, gsem1, gsem2, gsem3)
    ssems = (ssem0, ssem1, ssem2, ssem3)

    def start_gather(ch):
        return pltpu.async_copy(
            table_hbm.at[idx_all.at[pl.ds(ch * CHUNK, CHUNK)]],
            bufs[ch % NBUF], gsems[ch % NBUF])

    AHEAD = 2
    gcps = [None] * NCHUNK
    scps = [None] * NCHUNK
    for ch in range(min(AHEAD, NCHUNK)):
        gcps[ch] = start_gather(ch)
    for ch in range(NCHUNK):
        b = ch % NBUF
        gcps[ch].wait()
        scps[ch] = pltpu.async_copy(
            bufs[b], out_hbm.at[pl.ds(base + ch * CHUNK, CHUNK)], ssems[b])
        nxt = ch + AHEAD
        if nxt < NCHUNK:
            if nxt - NBUF >= 0:
                scps[nxt - NBUF].wait()
            gcps[nxt] = start_gather(nxt)
    for ch in range(max(0, NCHUNK - NBUF), NCHUNK):
        if scps[ch] is not None:
            scps[ch].wait()


def _sc_gather(seq_flat_seg, table):
    mesh = plsc.VectorSubcoreMesh(core_axis_name="c", subcore_axis_name="s")
    return pl.kernel(
        _sc_gather_body,
        mesh=mesh,
        out_type=jax.ShapeDtypeStruct((ROWS_SEG, D), jnp.float32),
        scratch_types=(
            [pltpu.VMEM((ROWS_PER_W,), jnp.int32)]
            + [pltpu.VMEM((CHUNK, D), jnp.float32) for _ in range(NBUF)]
            + [pltpu.SemaphoreType.DMA for _ in range(2 * NBUF)]
        ),
    )(seq_flat_seg, table)


LC = 5


def _gru_body(emb_ref, h0_ref, wih_ref, whh_ref, bih_ref, bhh_ref, out_ref,
              h_ref):
    l = pl.program_id(0)

    @pl.when(l == 0)
    def _():
        h_ref[...] = h0_ref[...]

    h = h_ref[...]
    for t in range(LC):
        x_t = emb_ref[t].astype(jnp.bfloat16)
        gi = (
            jnp.dot(x_t, wih_ref[...], preferred_element_type=jnp.float32)
            + bih_ref[...]
        )
        gh = (
            jnp.dot(h.astype(jnp.bfloat16), whh_ref[...],
                    preferred_element_type=jnp.float32)
            + bhh_ref[...]
        )
        r = 0.5 + 0.5 * jnp.tanh(0.5 * (gi[:, :D] + gh[:, :D]))
        z = 0.5 + 0.5 * jnp.tanh(0.5 * (gi[:, D:2 * D] + gh[:, D:2 * D]))
        n = jnp.tanh(gi[:, 2 * D:] + r * gh[:, 2 * D:])
        h = n + z * (h - n)

    h_ref[...] = h
    out_ref[...] = h


def _gru(emb_lbd, h0, wih_t, whh_t, b_ih2, b_hh2):
    return pl.pallas_call(
        _gru_body,
        grid=(LSEG // LC,),
        in_specs=[
            pl.BlockSpec((LC, B, D), lambda l: (l, 0, 0)),
            pl.BlockSpec((B, D), lambda l: (0, 0)),
            pl.BlockSpec((D, 3 * D), lambda l: (0, 0)),
            pl.BlockSpec((D, 3 * D), lambda l: (0, 0)),
            pl.BlockSpec((1, 3 * D), lambda l: (0, 0)),
            pl.BlockSpec((1, 3 * D), lambda l: (0, 0)),
        ],
        out_specs=pl.BlockSpec((B, D), lambda l: (0, 0)),
        out_shape=jax.ShapeDtypeStruct((B, D), jnp.float32),
        scratch_shapes=[
            pltpu.VMEM((B, D), jnp.float32),
        ],
    )(emb_lbd, h0, wih_t, whh_t, b_ih2, b_hh2)


@jax.jit
def kernel(item_seq, item_table, W_ih, W_hh, b_ih, b_hh):
    seq = jnp.where(item_seq == -1, PAD_IDX, item_seq).astype(jnp.int32)
    seq_t = seq.T.reshape(L * B)
    wih_t = W_ih.T.astype(jnp.bfloat16)
    whh_t = W_hh.T.astype(jnp.bfloat16)
    b_ih2 = b_ih.reshape(1, 3 * D)
    b_hh2 = b_hh.reshape(1, 3 * D)
    embs = [
        _sc_gather(lax.slice(seq_t, (s * ROWS_SEG,), ((s + 1) * ROWS_SEG,)),
                   item_table)
        for s in range(NSEG)
    ]
    h = jnp.zeros((B, D), jnp.float32)
    for s in range(NSEG):
        h = _gru(embs[s].reshape(LSEG, B, D), h, wih_t, whh_t, b_ih2, b_hh2)
    return h
